# baseline (speedup 1.0000x reference)
import functools

import jax
import jax.numpy as jnp
from jax.experimental import pallas as pl
from jax.experimental.pallas import tpu as pltpu

H, S, D = 12, 2048, 64
BLK = 64
NBLK = S // BLK
NSEL = 8
SCALE = 1.0 / (D ** 0.5)
SG = NSEL * BLK
NW = 2 * D
QB = 8
QG = NBLK // QB


def _prep_kernel(qt_ref, kt_ref, vt_ref, w_ref, bt_ref,
                 sel_ref, k16_ref, va16_ref, kvx_ref, kvt_ref):
    h = pl.program_id(0)
    qt = qt_ref[0, 0]
    kt = kt_ref[0, 0]
    vt = vt_ref[0, 0]
    w = w_ref[...]
    b = bt_ref[...]

    kh = jnp.transpose(kt)
    qh = jnp.transpose(qt)
    k16_ref[0] = kh.astype(jnp.bfloat16)
    va16_ref[0, :, 0:D] = jnp.transpose(vt).astype(jnp.bfloat16)
    va16_ref[0, :, D:D + 1] = jnp.ones((S, 1), jnp.bfloat16)
    va16_ref[0, :, D + 1:NW] = jnp.zeros((S, NW - D - 1), jnp.bfloat16)

    kproj_t = jax.lax.dot_general(w.astype(jnp.bfloat16),
                                  kt.astype(jnp.bfloat16),
                                  (((1,), (0,)), ((), ())),
                                  preferred_element_type=jnp.float32)
    kproj_t = kproj_t + b
    mx = jnp.max(kproj_t, axis=0, keepdims=True)
    ex = jnp.exp(kproj_t - mx)
    klt = (ex / jnp.sum(ex, axis=0, keepdims=True)).astype(jnp.bfloat16)

    va16 = va16_ref[0]
    for n in range(NBLK):
        kvx_ref[0, n] = jax.lax.dot_general(
            klt[:, n * BLK:(n + 1) * BLK], va16[n * BLK:(n + 1) * BLK, :],
            (((1,), (0,)), ((), ())),
            preferred_element_type=jnp.float32)
    kvt_ref[0] = jax.lax.dot_general(klt, va16, (((1,), (0,)), ((), ())),
                                     preferred_element_type=jnp.float32)

    qb = jnp.mean(qh.reshape(NBLK, BLK, D), axis=1)
    kb = jnp.mean(kh.reshape(NBLK, BLK, D), axis=1)
    scores = jax.lax.dot_general(qb, kb, (((1,), (1,)), ((), ())),
                                 preferred_element_type=jnp.float32) * SCALE

    colid = jax.lax.broadcasted_iota(jnp.int32, (NBLK, NBLK), 1)
    colid8 = jax.lax.broadcasted_iota(jnp.int32, (NBLK, NSEL), 1)
    cur = scores
    selmat = jnp.zeros((NBLK, NSEL), jnp.int32)
    for t in range(NSEL):
        mx2 = jnp.max(cur, axis=1, keepdims=True)
        cand = jnp.where(cur == mx2, colid, NBLK)
        amin = jnp.min(cand, axis=1, keepdims=True)
        selmat = selmat + amin * (colid8 == t).astype(jnp.int32)
        cur = jnp.where(colid == amin, -1e30, cur)
    sel_ref[0] = selmat


def _attn_kernel(sel_ref, qt_ref, k16_ref, va16_ref, kvx_ref, kvt_ref,
                 w_ref, b_ref, o_ref, kg_ref, vg_ref):
    h = pl.program_id(0)
    qi = pl.program_id(1)
    w16 = w_ref[...].astype(jnp.bfloat16)
    b = b_ref[...]

    kvt = kvt_ref[0]
    for j in range(QB):
        base = (h * NBLK + qi * QB + j) * NSEL
        kv_eff = kvt
        for t in range(NSEL):
            idx = sel_ref[base + t]
            kg_ref[j, pl.ds(t * BLK, BLK), :] = k16_ref[0, pl.ds(idx * BLK, BLK), :]
            vg_ref[j, pl.ds(t * BLK, BLK), :] = va16_ref[0, pl.ds(idx * BLK, BLK), :]
            kv_eff = kv_eff - kvx_ref[0, idx]

        qtj = qt_ref[0, 0, :, pl.ds(j * BLK, BLK)].astype(jnp.bfloat16)
        qproj = jax.lax.dot_general(qtj, w16, (((0,), (1,)), ((), ())),
                                    preferred_element_type=jnp.float32) + b
        ql = jax.nn.softmax(qproj, axis=-1)

        s = jax.lax.dot_general(qtj, kg_ref[j], (((0,), (1,)), ((), ())),
                                preferred_element_type=jnp.float32) * SCALE
        p = jnp.exp(s)
        big = jnp.dot(p.astype(jnp.bfloat16), vg_ref[j],
                      preferred_element_type=jnp.float32)
        big = big + jnp.dot(ql.astype(jnp.bfloat16), kv_eff.astype(jnp.bfloat16),
                            preferred_element_type=jnp.float32)
        o_ref[0, 0, pl.ds(j * BLK, BLK), :] = (
            big[:, 0:D] / (big[:, D:D + 1] + 1e-6))


@jax.jit
def _run(q, k, v, w, b):
    b2 = b.reshape(1, D)
    bt = b.reshape(D, 1)
    qt = jnp.transpose(q, (0, 1, 3, 2))
    kt = jnp.transpose(k, (0, 1, 3, 2))
    vt = jnp.transpose(v, (0, 1, 3, 2))

    sel, k16, va16, kvx, kvt = pl.pallas_call(
        _prep_kernel,
        grid=(H,),
        in_specs=[
            pl.BlockSpec((1, 1, D, S), lambda h: (0, h, 0, 0)),
            pl.BlockSpec((1, 1, D, S), lambda h: (0, h, 0, 0)),
            pl.BlockSpec((1, 1, D, S), lambda h: (0, h, 0, 0)),
            pl.BlockSpec((D, D), lambda h: (0, 0)),
            pl.BlockSpec((D, 1), lambda h: (0, 0)),
        ],
        out_specs=[
            pl.BlockSpec((1, NBLK, NSEL), lambda h: (h, 0, 0)),
            pl.BlockSpec((1, S, D), lambda h: (h, 0, 0)),
            pl.BlockSpec((1, S, NW), lambda h: (h, 0, 0)),
            pl.BlockSpec((1, NBLK, D, NW), lambda h: (h, 0, 0, 0)),
            pl.BlockSpec((1, D, NW), lambda h: (h, 0, 0)),
        ],
        out_shape=[
            jax.ShapeDtypeStruct((H, NBLK, NSEL), jnp.int32),
            jax.ShapeDtypeStruct((H, S, D), jnp.bfloat16),
            jax.ShapeDtypeStruct((H, S, NW), jnp.bfloat16),
            jax.ShapeDtypeStruct((H, NBLK, D, NW), jnp.float32),
            jax.ShapeDtypeStruct((H, D, NW), jnp.float32),
        ],
    )(qt, kt, vt, w, bt)

    out = pl.pallas_call(
        _attn_kernel,
        grid_spec=pltpu.PrefetchScalarGridSpec(
            num_scalar_prefetch=1,
            grid=(H, QG),
            in_specs=[
                pl.BlockSpec((1, 1, D, QB * BLK), lambda h, qi, sel: (0, h, 0, qi)),
                pl.BlockSpec((1, S, D), lambda h, qi, sel: (h, 0, 0)),
                pl.BlockSpec((1, S, NW), lambda h, qi, sel: (h, 0, 0)),
                pl.BlockSpec((1, NBLK, D, NW), lambda h, qi, sel: (h, 0, 0, 0)),
                pl.BlockSpec((1, D, NW), lambda h, qi, sel: (h, 0, 0)),
                pl.BlockSpec((D, D), lambda h, qi, sel: (0, 0)),
                pl.BlockSpec((1, D), lambda h, qi, sel: (0, 0)),
            ],
            out_specs=pl.BlockSpec((1, 1, QB * BLK, D),
                                   lambda h, qi, sel: (0, h, qi, 0)),
            scratch_shapes=[
                pltpu.VMEM((QB, SG, D), jnp.bfloat16),
                pltpu.VMEM((QB, SG, NW), jnp.bfloat16),
            ],
        ),
        out_shape=jax.ShapeDtypeStruct((1, H, S, D), jnp.float32),
    )(sel.reshape(-1), qt, k16, va16, kvx, kvt, w, b2)

    return out


def kernel(q, k, v, W_l, b_l):
    return _run(q, k, v, W_l, b_l)


# speedup vs baseline: 1.0561x; 1.0561x over previous
import functools

import jax
import jax.numpy as jnp
from jax.experimental import pallas as pl
from jax.experimental.pallas import tpu as pltpu

H, S, D = 12, 2048, 64
BLK = 64
NBLK = S // BLK
NSEL = 8
SCALE = 1.0 / (D ** 0.5)
SG = NSEL * BLK
NW = 2 * D
QB = 16
QG = NBLK // QB


def _prep_kernel(qt_ref, kt_ref, vt_ref, w_ref, bt_ref,
                 sel_ref, k16_ref, va16_ref, kvx_ref, kvt_ref):
    h = pl.program_id(0)
    qt = qt_ref[0, 0]
    kt = kt_ref[0, 0]
    vt = vt_ref[0, 0]
    w = w_ref[...]
    b = bt_ref[...]

    kh = jnp.transpose(kt)
    qh = jnp.transpose(qt)
    k16_ref[0] = kh.astype(jnp.bfloat16)
    va16_ref[0, :, 0:D] = jnp.transpose(vt).astype(jnp.bfloat16)
    va16_ref[0, :, D:D + 1] = jnp.ones((S, 1), jnp.bfloat16)
    va16_ref[0, :, D + 1:NW] = jnp.zeros((S, NW - D - 1), jnp.bfloat16)

    kproj_t = jax.lax.dot_general(w.astype(jnp.bfloat16),
                                  kt.astype(jnp.bfloat16),
                                  (((1,), (0,)), ((), ())),
                                  preferred_element_type=jnp.float32)
    kproj_t = kproj_t + b
    mx = jnp.max(kproj_t, axis=0, keepdims=True)
    ex = jnp.exp(kproj_t - mx)
    klt = (ex / jnp.sum(ex, axis=0, keepdims=True)).astype(jnp.bfloat16)

    va16 = va16_ref[0]
    for n in range(NBLK):
        kvx_ref[0, n] = jax.lax.dot_general(
            klt[:, n * BLK:(n + 1) * BLK], va16[n * BLK:(n + 1) * BLK, :],
            (((1,), (0,)), ((), ())),
            preferred_element_type=jnp.float32)
    kvt_ref[0] = jax.lax.dot_general(klt, va16, (((1,), (0,)), ((), ())),
                                     preferred_element_type=jnp.float32)

    qb = jnp.mean(qh.reshape(NBLK, BLK, D), axis=1)
    kb = jnp.mean(kh.reshape(NBLK, BLK, D), axis=1)
    scores = jax.lax.dot_general(qb, kb, (((1,), (1,)), ((), ())),
                                 preferred_element_type=jnp.float32) * SCALE

    colid = jax.lax.broadcasted_iota(jnp.int32, (NBLK, NBLK), 1)
    colid8 = jax.lax.broadcasted_iota(jnp.int32, (NBLK, NSEL), 1)
    cur = scores
    selmat = jnp.zeros((NBLK, NSEL), jnp.int32)
    for t in range(NSEL):
        mx2 = jnp.max(cur, axis=1, keepdims=True)
        cand = jnp.where(cur == mx2, colid, NBLK)
        amin = jnp.min(cand, axis=1, keepdims=True)
        selmat = selmat + amin * (colid8 == t).astype(jnp.int32)
        cur = jnp.where(colid == amin, -1e30, cur)
    sel_ref[0] = selmat


def _attn_kernel(sel_ref, qt_ref, k16_ref, va16_ref, kvx_ref, kvt_ref,
                 w_ref, b_ref, o_ref, kg_ref, vg_ref):
    h = pl.program_id(0)
    qi = pl.program_id(1)
    w16 = w_ref[...].astype(jnp.bfloat16)
    b = b_ref[...]

    kvt = kvt_ref[0]
    for j in range(QB):
        base = (h * NBLK + qi * QB + j) * NSEL
        kv_eff = kvt
        for t in range(NSEL):
            idx = sel_ref[base + t]
            kg_ref[j, pl.ds(t * BLK, BLK), :] = k16_ref[0, pl.ds(idx * BLK, BLK), :]
            vg_ref[j, pl.ds(t * BLK, BLK), :] = va16_ref[0, pl.ds(idx * BLK, BLK), :]
            kv_eff = kv_eff - kvx_ref[0, idx]

        qtj = qt_ref[0, 0, :, pl.ds(j * BLK, BLK)].astype(jnp.bfloat16)
        qproj = jax.lax.dot_general(qtj, w16, (((0,), (1,)), ((), ())),
                                    preferred_element_type=jnp.float32) + b
        ql = jax.nn.softmax(qproj, axis=-1)

        s = jax.lax.dot_general(qtj, kg_ref[j], (((0,), (1,)), ((), ())),
                                preferred_element_type=jnp.float32) * SCALE
        p = jnp.exp(s)
        big = jnp.dot(p.astype(jnp.bfloat16), vg_ref[j],
                      preferred_element_type=jnp.float32)
        big = big + jnp.dot(ql.astype(jnp.bfloat16), kv_eff.astype(jnp.bfloat16),
                            preferred_element_type=jnp.float32)
        o_ref[0, 0, pl.ds(j * BLK, BLK), :] = (
            big[:, 0:D] / (big[:, D:D + 1] + 1e-6))


@jax.jit
def _run(q, k, v, w, b):
    b2 = b.reshape(1, D)
    bt = b.reshape(D, 1)
    qt = jnp.transpose(q, (0, 1, 3, 2))
    kt = jnp.transpose(k, (0, 1, 3, 2))
    vt = jnp.transpose(v, (0, 1, 3, 2))

    sel, k16, va16, kvx, kvt = pl.pallas_call(
        _prep_kernel,
        grid=(H,),
        in_specs=[
            pl.BlockSpec((1, 1, D, S), lambda h: (0, h, 0, 0)),
            pl.BlockSpec((1, 1, D, S), lambda h: (0, h, 0, 0)),
            pl.BlockSpec((1, 1, D, S), lambda h: (0, h, 0, 0)),
            pl.BlockSpec((D, D), lambda h: (0, 0)),
            pl.BlockSpec((D, 1), lambda h: (0, 0)),
        ],
        out_specs=[
            pl.BlockSpec((1, NBLK, NSEL), lambda h: (h, 0, 0)),
            pl.BlockSpec((1, S, D), lambda h: (h, 0, 0)),
            pl.BlockSpec((1, S, NW), lambda h: (h, 0, 0)),
            pl.BlockSpec((1, NBLK, D, NW), lambda h: (h, 0, 0, 0)),
            pl.BlockSpec((1, D, NW), lambda h: (h, 0, 0)),
        ],
        out_shape=[
            jax.ShapeDtypeStruct((H, NBLK, NSEL), jnp.int32),
            jax.ShapeDtypeStruct((H, S, D), jnp.bfloat16),
            jax.ShapeDtypeStruct((H, S, NW), jnp.bfloat16),
            jax.ShapeDtypeStruct((H, NBLK, D, NW), jnp.float32),
            jax.ShapeDtypeStruct((H, D, NW), jnp.float32),
        ],
    )(qt, kt, vt, w, bt)

    out = pl.pallas_call(
        _attn_kernel,
        grid_spec=pltpu.PrefetchScalarGridSpec(
            num_scalar_prefetch=1,
            grid=(H, QG),
            in_specs=[
                pl.BlockSpec((1, 1, D, QB * BLK), lambda h, qi, sel: (0, h, 0, qi)),
                pl.BlockSpec((1, S, D), lambda h, qi, sel: (h, 0, 0)),
                pl.BlockSpec((1, S, NW), lambda h, qi, sel: (h, 0, 0)),
                pl.BlockSpec((1, NBLK, D, NW), lambda h, qi, sel: (h, 0, 0, 0)),
                pl.BlockSpec((1, D, NW), lambda h, qi, sel: (h, 0, 0)),
                pl.BlockSpec((D, D), lambda h, qi, sel: (0, 0)),
                pl.BlockSpec((1, D), lambda h, qi, sel: (0, 0)),
            ],
            out_specs=pl.BlockSpec((1, 1, QB * BLK, D),
                                   lambda h, qi, sel: (0, h, qi, 0)),
            scratch_shapes=[
                pltpu.VMEM((QB, SG, D), jnp.bfloat16),
                pltpu.VMEM((QB, SG, NW), jnp.bfloat16),
            ],
        ),
        out_shape=jax.ShapeDtypeStruct((1, H, S, D), jnp.float32),
    )(sel.reshape(-1), qt, k16, va16, kvx, kvt, w, b2)

    return out


def kernel(q, k, v, W_l, b_l):
    return _run(q, k, v, W_l, b_l)


# speedup vs baseline: 1.0771x; 1.0199x over previous
import functools

import jax
import jax.numpy as jnp
from jax.experimental import pallas as pl
from jax.experimental.pallas import tpu as pltpu

H, S, D = 12, 2048, 64
BLK = 64
NBLK = S // BLK
NSEL = 8
SCALE = 1.0 / (D ** 0.5)
SG = NSEL * BLK
NW = 2 * D
QB = 32
QG = NBLK // QB


def _prep_kernel(qt_ref, kt_ref, vt_ref, w_ref, bt_ref,
                 sel_ref, k16_ref, va16_ref, kvx_ref, kvt_ref):
    h = pl.program_id(0)
    qt = qt_ref[0, 0]
    kt = kt_ref[0, 0]
    vt = vt_ref[0, 0]
    w = w_ref[...]
    b = bt_ref[...]

    kh = jnp.transpose(kt)
    qh = jnp.transpose(qt)
    k16_ref[0] = kh.astype(jnp.bfloat16)
    va16_ref[0, :, 0:D] = jnp.transpose(vt).astype(jnp.bfloat16)
    va16_ref[0, :, D:D + 1] = jnp.ones((S, 1), jnp.bfloat16)
    va16_ref[0, :, D + 1:NW] = jnp.zeros((S, NW - D - 1), jnp.bfloat16)

    kproj_t = jax.lax.dot_general(w.astype(jnp.bfloat16),
                                  kt.astype(jnp.bfloat16),
                                  (((1,), (0,)), ((), ())),
                                  preferred_element_type=jnp.float32)
    kproj_t = kproj_t + b
    mx = jnp.max(kproj_t, axis=0, keepdims=True)
    ex = jnp.exp(kproj_t - mx)
    klt = (ex / jnp.sum(ex, axis=0, keepdims=True)).astype(jnp.bfloat16)

    va16 = va16_ref[0]
    for n in range(NBLK):
        kvx_ref[0, n] = jax.lax.dot_general(
            klt[:, n * BLK:(n + 1) * BLK], va16[n * BLK:(n + 1) * BLK, :],
            (((1,), (0,)), ((), ())),
            preferred_element_type=jnp.float32)
    kvt_ref[0] = jax.lax.dot_general(klt, va16, (((1,), (0,)), ((), ())),
                                     preferred_element_type=jnp.float32)

    qb = jnp.mean(qh.reshape(NBLK, BLK, D), axis=1)
    kb = jnp.mean(kh.reshape(NBLK, BLK, D), axis=1)
    scores = jax.lax.dot_general(qb, kb, (((1,), (1,)), ((), ())),
                                 preferred_element_type=jnp.float32) * SCALE

    colid = jax.lax.broadcasted_iota(jnp.int32, (NBLK, NBLK), 1)
    colid8 = jax.lax.broadcasted_iota(jnp.int32, (NBLK, NSEL), 1)
    cur = scores
    selmat = jnp.zeros((NBLK, NSEL), jnp.int32)
    for t in range(NSEL):
        mx2 = jnp.max(cur, axis=1, keepdims=True)
        cand = jnp.where(cur == mx2, colid, NBLK)
        amin = jnp.min(cand, axis=1, keepdims=True)
        selmat = selmat + amin * (colid8 == t).astype(jnp.int32)
        cur = jnp.where(colid == amin, -1e30, cur)
    sel_ref[0] = selmat


def _attn_kernel(sel_ref, qt_ref, k16_ref, va16_ref, kvx_ref, kvt_ref,
                 w_ref, b_ref, o_ref, kg_ref, vg_ref):
    h = pl.program_id(0)
    qi = pl.program_id(1)
    w16 = w_ref[...].astype(jnp.bfloat16)
    b = b_ref[...]

    kvt = kvt_ref[0]
    for j in range(QB):
        base = (h * NBLK + qi * QB + j) * NSEL
        kv_eff = kvt
        for t in range(NSEL):
            idx = sel_ref[base + t]
            kg_ref[j, pl.ds(t * BLK, BLK), :] = k16_ref[0, pl.ds(idx * BLK, BLK), :]
            vg_ref[j, pl.ds(t * BLK, BLK), :] = va16_ref[0, pl.ds(idx * BLK, BLK), :]
            kv_eff = kv_eff - kvx_ref[0, idx]

        qtj = qt_ref[0, 0, :, pl.ds(j * BLK, BLK)].astype(jnp.bfloat16)
        qproj = jax.lax.dot_general(qtj, w16, (((0,), (1,)), ((), ())),
                                    preferred_element_type=jnp.float32) + b
        ql = jax.nn.softmax(qproj, axis=-1)

        s = jax.lax.dot_general(qtj, kg_ref[j], (((0,), (1,)), ((), ())),
                                preferred_element_type=jnp.float32) * SCALE
        p = jnp.exp(s)
        big = jnp.dot(p.astype(jnp.bfloat16), vg_ref[j],
                      preferred_element_type=jnp.float32)
        big = big + jnp.dot(ql.astype(jnp.bfloat16), kv_eff.astype(jnp.bfloat16),
                            preferred_element_type=jnp.float32)
        o_ref[0, 0, pl.ds(j * BLK, BLK), :] = (
            big[:, 0:D] / (big[:, D:D + 1] + 1e-6))


@jax.jit
def _run(q, k, v, w, b):
    b2 = b.reshape(1, D)
    bt = b.reshape(D, 1)
    qt = jnp.transpose(q, (0, 1, 3, 2))
    kt = jnp.transpose(k, (0, 1, 3, 2))
    vt = jnp.transpose(v, (0, 1, 3, 2))

    sel, k16, va16, kvx, kvt = pl.pallas_call(
        _prep_kernel,
        grid=(H,),
        in_specs=[
            pl.BlockSpec((1, 1, D, S), lambda h: (0, h, 0, 0)),
            pl.BlockSpec((1, 1, D, S), lambda h: (0, h, 0, 0)),
            pl.BlockSpec((1, 1, D, S), lambda h: (0, h, 0, 0)),
            pl.BlockSpec((D, D), lambda h: (0, 0)),
            pl.BlockSpec((D, 1), lambda h: (0, 0)),
        ],
        out_specs=[
            pl.BlockSpec((1, NBLK, NSEL), lambda h: (h, 0, 0)),
            pl.BlockSpec((1, S, D), lambda h: (h, 0, 0)),
            pl.BlockSpec((1, S, NW), lambda h: (h, 0, 0)),
            pl.BlockSpec((1, NBLK, D, NW), lambda h: (h, 0, 0, 0)),
            pl.BlockSpec((1, D, NW), lambda h: (h, 0, 0)),
        ],
        out_shape=[
            jax.ShapeDtypeStruct((H, NBLK, NSEL), jnp.int32),
            jax.ShapeDtypeStruct((H, S, D), jnp.bfloat16),
            jax.ShapeDtypeStruct((H, S, NW), jnp.bfloat16),
            jax.ShapeDtypeStruct((H, NBLK, D, NW), jnp.float32),
            jax.ShapeDtypeStruct((H, D, NW), jnp.float32),
        ],
    )(qt, kt, vt, w, bt)

    out = pl.pallas_call(
        _attn_kernel,
        grid_spec=pltpu.PrefetchScalarGridSpec(
            num_scalar_prefetch=1,
            grid=(H, QG),
            in_specs=[
                pl.BlockSpec((1, 1, D, QB * BLK), lambda h, qi, sel: (0, h, 0, qi)),
                pl.BlockSpec((1, S, D), lambda h, qi, sel: (h, 0, 0)),
                pl.BlockSpec((1, S, NW), lambda h, qi, sel: (h, 0, 0)),
                pl.BlockSpec((1, NBLK, D, NW), lambda h, qi, sel: (h, 0, 0, 0)),
                pl.BlockSpec((1, D, NW), lambda h, qi, sel: (h, 0, 0)),
                pl.BlockSpec((D, D), lambda h, qi, sel: (0, 0)),
                pl.BlockSpec((1, D), lambda h, qi, sel: (0, 0)),
            ],
            out_specs=pl.BlockSpec((1, 1, QB * BLK, D),
                                   lambda h, qi, sel: (0, h, qi, 0)),
            scratch_shapes=[
                pltpu.VMEM((QB, SG, D), jnp.bfloat16),
                pltpu.VMEM((QB, SG, NW), jnp.bfloat16),
            ],
        ),
        out_shape=jax.ShapeDtypeStruct((1, H, S, D), jnp.float32),
    )(sel.reshape(-1), qt, k16, va16, kvx, kvt, w, b2)

    return out


def kernel(q, k, v, W_l, b_l):
    return _run(q, k, v, W_l, b_l)
